# baseline (speedup 1.0000x reference)
import functools

import jax
import jax.numpy as jnp
from jax import lax
from jax.experimental import pallas as pl
from jax.experimental.pallas import tpu as pltpu
from jax.experimental.pallas import tpu_sc as plsc

N = 4096
D = 1024
H = 4096
E = 8
TOPK = 2
A = N * TOPK
R = 384
NBLK = -(-(A + E * (R - 1)) // R)
PADN = NBLK * R
HT = 2048

NC, NS, L = 2, 16, 16
NW = NC * NS
CHUNK = A // NW


def _gate_kernel(x_ref, wg_ref, bg_ref, w_ref, i_ref):
    scores = jnp.dot(x_ref[...], wg_ref[...],
                     preferred_element_type=jnp.float32) + bg_ref[...]
    lane = jax.lax.broadcasted_iota(jnp.int32, scores.shape, 1)
    m1 = jnp.max(scores, axis=1, keepdims=True)
    a1 = jnp.argmax(scores, axis=1).reshape(-1, 1)
    masked = jnp.where(lane == a1, -jnp.inf, scores)
    m2 = jnp.max(masked, axis=1, keepdims=True)
    a2 = jnp.argmax(masked, axis=1).reshape(-1, 1)
    z = jnp.exp(m2 - m1)
    w_ref[...] = jnp.concatenate([1.0 / (1.0 + z), z / (1.0 + z)], axis=1)
    i_ref[...] = jnp.concatenate([a1, a2], axis=1)


def _gate(x2d, Wg, bg):
    TB = 1024
    return pl.pallas_call(
        _gate_kernel,
        grid=(N // TB,),
        in_specs=[
            pl.BlockSpec((TB, D), lambda t: (t, 0)),
            pl.BlockSpec((D, E), lambda t: (0, 0)),
            pl.BlockSpec((E,), lambda t: (0,)),
        ],
        out_specs=[
            pl.BlockSpec((TB, TOPK), lambda t: (t, 0)),
            pl.BlockSpec((TB, TOPK), lambda t: (t, 0)),
        ],
        out_shape=[
            jax.ShapeDtypeStruct((N, TOPK), jnp.float32),
            jax.ShapeDtypeStruct((N, TOPK), jnp.int32),
        ],
    )(x2d, Wg, bg)


def _dispatch_body(idx_hbm, x_hbm, xs_hbm, dest_hbm, gs_hbm,
                   idx_v, cnt_v, gs_v, dchunk_v, tok_v, rows_v, sem):
    wid = lax.axis_index("s") * NC + lax.axis_index("c")
    base = wid * CHUNK
    lanes = lax.iota(jnp.int32, L)
    onesf = jnp.ones((L,), jnp.int32)

    pltpu.sync_copy(idx_hbm, idx_v)

    def hist_step(j, carry):
        tot, pre = carry
        v = idx_v[pl.ds(j * L, L)]
        h = jnp.zeros((L,), jnp.int32)
        for e in range(E):
            c = jnp.sum(jnp.where(v == e, 1, 0).astype(jnp.int32))
            h = jnp.where(lanes == e, c, h)
        inpre = jnp.where(j < wid * (CHUNK // L), 1, 0)
        return tot + h, pre + h * inpre

    tot, pre = lax.fori_loop(
        0, A // L, hist_step,
        (jnp.zeros((L,), jnp.int32), jnp.zeros((L,), jnp.int32)))

    padded = (tot + (R - 1)) & ~(R - 1)
    gs = plsc.cumsum(padded) - padded
    gs_v[...] = gs
    cnt_v[...] = gs + pre

    @pl.when(wid == 0)
    def _():
        pltpu.sync_copy(gs_v, gs_hbm)

    for j in range(CHUNK // L):
        v = idx_v[pl.ds(base + j * L, L)]
        rank = jnp.zeros((L,), jnp.int32)
        add = jnp.zeros((L,), jnp.int32)
        for e in range(E):
            m = v == e
            mi = jnp.where(m, 1, 0).astype(jnp.int32)
            s = plsc.cumsum(mi)
            rank = jnp.where(m, s - 1, rank)
            add = jnp.where(lanes == e, jnp.sum(mi), add)
        dest = plsc.load_gather(cnt_v, [v]) + rank
        cnt_v[...] = cnt_v[...] + add
        dchunk_v[j // 4, pl.ds((j % 4) * L, L)] = dest

    for c in range(4):
        pltpu.sync_copy(dchunk_v.at[c], dest_hbm.at[pl.ds(base + c * 64, 64)])
        for j in range(4):
            a0 = base + c * 64 + j * L
            tok_v[pl.ds(j * L, L)] = (a0 + lanes) // TOPK
        pltpu.async_copy(x_hbm.at[tok_v], rows_v, sem).wait()
        pltpu.async_copy(rows_v, xs_hbm.at[dchunk_v.at[c]], sem).wait()


def _dispatch(idxflat, x2d):
    mesh = plsc.VectorSubcoreMesh(core_axis_name="c", subcore_axis_name="s")
    f = pl.kernel(
        _dispatch_body,
        out_type=(
            jax.ShapeDtypeStruct((PADN, D), jnp.float32),
            jax.ShapeDtypeStruct((A,), jnp.int32),
            jax.ShapeDtypeStruct((L,), jnp.int32),
        ),
        mesh=mesh,
        compiler_params=pltpu.CompilerParams(needs_layout_passes=False),
        scratch_types=[
            pltpu.VMEM((A,), jnp.int32),
            pltpu.VMEM((L,), jnp.int32),
            pltpu.VMEM((L,), jnp.int32),
            pltpu.VMEM((4, 64), jnp.int32),
            pltpu.VMEM((64,), jnp.int32),
            pltpu.VMEM((64, D), jnp.float32),
            pltpu.SemaphoreType.DMA,
        ],
    )
    return f(idxflat, x2d)


def _gmm_kernel(s_ref, xs_ref, w1_ref, b1_ref, w2_ref, b2_ref, out_ref):
    h = pl.program_id(1)

    @pl.when(h == 0)
    def _():
        out_ref[...] = jnp.broadcast_to(b2_ref[0], out_ref.shape)

    hpart = jnp.maximum(
        jnp.dot(xs_ref[...], w1_ref[0], preferred_element_type=jnp.float32)
        + b1_ref[0], 0.0)
    out_ref[...] += jnp.dot(hpart, w2_ref[0],
                            preferred_element_type=jnp.float32)


def _e_of(b, s_ref):
    val = b * R
    e = jnp.int32(0)
    for ee in range(1, E):
        e += jnp.where(val >= s_ref[ee], 1, 0).astype(jnp.int32)
    return e


def _gmm(gs, Xs, W1, b1, W2, b2):
    grid = (NBLK, H // HT)
    grid_spec = pltpu.PrefetchScalarGridSpec(
        num_scalar_prefetch=1,
        grid=grid,
        in_specs=[
            pl.BlockSpec((R, D), lambda b, h, s: (b, 0)),
            pl.BlockSpec((1, D, HT), lambda b, h, s: (_e_of(b, s), 0, h)),
            pl.BlockSpec((1, 1, HT), lambda b, h, s: (_e_of(b, s), 0, h)),
            pl.BlockSpec((1, HT, D), lambda b, h, s: (_e_of(b, s), h, 0)),
            pl.BlockSpec((1, 1, D), lambda b, h, s: (_e_of(b, s), 0, 0)),
        ],
        out_specs=pl.BlockSpec((R, D), lambda b, h, s: (b, 0)),
    )
    return pl.pallas_call(
        _gmm_kernel,
        grid_spec=grid_spec,
        out_shape=jax.ShapeDtypeStruct((PADN, D), jnp.float32),
    )(gs, Xs, W1, b1.reshape(E, 1, H), W2, b2.reshape(E, 1, D))


def _combine_body(ys_hbm, dest_hbm, w_hbm, out_hbm,
                  dest_v, w_v, ybuf_v, obuf_v, sem):
    wid = lax.axis_index("s") * NC + lax.axis_index("c")
    base = wid * CHUNK
    tok0 = wid * (N // NW)

    pltpu.sync_copy(dest_hbm.at[pl.ds(base, CHUNK)], dest_v)
    pltpu.sync_copy(w_hbm.at[pl.ds(base, CHUNK)], w_v)

    def chunk_step(c, _):
        pltpu.async_copy(ys_hbm.at[dest_v.at[pl.ds(c * 16, 16)]],
                         ybuf_v, sem).wait()
        for t in range(8):
            w0 = plsc.load_gather(w_v, [jnp.zeros((L,), jnp.int32)
                                        + (c * 16 + 2 * t)])
            w1 = plsc.load_gather(w_v, [jnp.zeros((L,), jnp.int32)
                                        + (c * 16 + 2 * t + 1)])
            for q in range(D // L):
                y0 = ybuf_v[2 * t, pl.ds(q * L, L)]
                y1 = ybuf_v[2 * t + 1, pl.ds(q * L, L)]
                obuf_v[t, pl.ds(q * L, L)] = w0 * y0 + w1 * y1
        pltpu.sync_copy(obuf_v, out_hbm.at[pl.ds(tok0 + c * 8, 8)])
        return 0

    lax.fori_loop(0, CHUNK // 16, chunk_step, 0)


def _combine(Ys, dest, wflat):
    mesh = plsc.VectorSubcoreMesh(core_axis_name="c", subcore_axis_name="s")
    f = pl.kernel(
        _combine_body,
        out_type=jax.ShapeDtypeStruct((N, D), jnp.float32),
        mesh=mesh,
        compiler_params=pltpu.CompilerParams(needs_layout_passes=False),
        scratch_types=[
            pltpu.VMEM((CHUNK,), jnp.int32),
            pltpu.VMEM((CHUNK,), jnp.float32),
            pltpu.VMEM((16, D), jnp.float32),
            pltpu.VMEM((8, D), jnp.float32),
            pltpu.SemaphoreType.DMA,
        ],
    )
    return f(Ys, dest, wflat)


def kernel(x, Wg, bg, W1, b1, W2, b2):
    B, S, _ = x.shape
    x2d = x.reshape(N, D)
    topw, topi = _gate(x2d, Wg, bg)
    Xs, dest, gs = _dispatch(topi.reshape(A), x2d)
    Ys = _gmm(gs, Xs, W1, b1, W2, b2)
    out = _combine(Ys, dest, topw.reshape(A))
    return out.reshape(B, S, D)


# speedup vs baseline: 1.0978x; 1.0978x over previous
import functools

import jax
import jax.numpy as jnp
from jax import lax
from jax.experimental import pallas as pl
from jax.experimental.pallas import tpu as pltpu
from jax.experimental.pallas import tpu_sc as plsc

N = 4096
D = 1024
H = 4096
E = 8
TOPK = 2
A = N * TOPK
R = 512
PADN = A + E * R
NBLK = PADN // R
HT = 2048

NC, NS, L = 2, 16, 16
NW = NC * NS
CHUNK = A // NW


def _gate_kernel(x_ref, wg_ref, bg_ref, w_ref, i_ref):
    scores = jnp.dot(x_ref[...], wg_ref[...],
                     preferred_element_type=jnp.float32) + bg_ref[...]
    lane = jax.lax.broadcasted_iota(jnp.int32, scores.shape, 1)
    m1 = jnp.max(scores, axis=1, keepdims=True)
    a1 = jnp.argmax(scores, axis=1).reshape(-1, 1)
    masked = jnp.where(lane == a1, -jnp.inf, scores)
    m2 = jnp.max(masked, axis=1, keepdims=True)
    a2 = jnp.argmax(masked, axis=1).reshape(-1, 1)
    z = jnp.exp(m2 - m1)
    w_ref[...] = jnp.concatenate([1.0 / (1.0 + z), z / (1.0 + z)], axis=1)
    i_ref[...] = jnp.concatenate([a1, a2], axis=1)


def _gate(x2d, Wg, bg):
    TB = 1024
    return pl.pallas_call(
        _gate_kernel,
        grid=(N // TB,),
        in_specs=[
            pl.BlockSpec((TB, D), lambda t: (t, 0)),
            pl.BlockSpec((D, E), lambda t: (0, 0)),
            pl.BlockSpec((E,), lambda t: (0,)),
        ],
        out_specs=[
            pl.BlockSpec((TB, TOPK), lambda t: (t, 0)),
            pl.BlockSpec((TB, TOPK), lambda t: (t, 0)),
        ],
        out_shape=[
            jax.ShapeDtypeStruct((N, TOPK), jnp.float32),
            jax.ShapeDtypeStruct((N, TOPK), jnp.int32),
        ],
    )(x2d, Wg, bg)


def _dispatch_body(idx_hbm, x_hbm, xs_hbm, dest_hbm, gs_hbm,
                   idx_v, cnt_v, gs_v, dchunk_v, tok_v, rows_v, sem):
    wid = lax.axis_index("s") * NC + lax.axis_index("c")
    base = wid * CHUNK
    lanes = lax.iota(jnp.int32, L)
    onesf = jnp.ones((L,), jnp.int32)

    pltpu.sync_copy(idx_hbm, idx_v)

    def hist_step(j, carry):
        tot, pre = carry
        v = idx_v[pl.ds(j * L, L)]
        h = jnp.zeros((L,), jnp.int32)
        for e in range(E):
            c = jnp.sum(jnp.where(v == e, 1, 0).astype(jnp.int32))
            h = jnp.where(lanes == e, c, h)
        inpre = jnp.where(j < wid * (CHUNK // L), 1, 0)
        return tot + h, pre + h * inpre

    tot, pre = lax.fori_loop(
        0, A // L, hist_step,
        (jnp.zeros((L,), jnp.int32), jnp.zeros((L,), jnp.int32)))

    padded = (tot + (R - 1)) & ~(R - 1)
    gs = plsc.cumsum(padded) - padded
    gs_v[...] = gs
    cnt_v[...] = gs + pre

    @pl.when(wid == 0)
    def _():
        pltpu.sync_copy(gs_v, gs_hbm)

    for j in range(CHUNK // L):
        v = idx_v[pl.ds(base + j * L, L)]
        rank = jnp.zeros((L,), jnp.int32)
        add = jnp.zeros((L,), jnp.int32)
        for e in range(E):
            m = v == e
            mi = jnp.where(m, 1, 0).astype(jnp.int32)
            s = plsc.cumsum(mi)
            rank = jnp.where(m, s - 1, rank)
            add = jnp.where(lanes == e, jnp.sum(mi), add)
        dest = plsc.load_gather(cnt_v, [v]) + rank
        cnt_v[...] = cnt_v[...] + add
        dchunk_v[j // 4, pl.ds((j % 4) * L, L)] = dest

    for c in range(4):
        pltpu.sync_copy(dchunk_v.at[c], dest_hbm.at[pl.ds(base + c * 64, 64)])
        for j in range(4):
            a0 = base + c * 64 + j * L
            tok_v[pl.ds(j * L, L)] = (a0 + lanes) // TOPK
        pltpu.async_copy(x_hbm.at[tok_v], rows_v, sem).wait()
        pltpu.async_copy(rows_v, xs_hbm.at[dchunk_v.at[c]], sem).wait()


def _dispatch(idxflat, x2d):
    mesh = plsc.VectorSubcoreMesh(core_axis_name="c", subcore_axis_name="s")
    f = pl.kernel(
        _dispatch_body,
        out_type=(
            jax.ShapeDtypeStruct((PADN, D), jnp.float32),
            jax.ShapeDtypeStruct((A,), jnp.int32),
            jax.ShapeDtypeStruct((L,), jnp.int32),
        ),
        mesh=mesh,
        compiler_params=pltpu.CompilerParams(needs_layout_passes=False),
        scratch_types=[
            pltpu.VMEM((A,), jnp.int32),
            pltpu.VMEM((L,), jnp.int32),
            pltpu.VMEM((L,), jnp.int32),
            pltpu.VMEM((4, 64), jnp.int32),
            pltpu.VMEM((64,), jnp.int32),
            pltpu.VMEM((64, D), jnp.float32),
            pltpu.SemaphoreType.DMA,
        ],
    )
    return f(idxflat, x2d)


def _gmm_kernel(s_ref, xs_ref, w1_ref, b1_ref, w2_ref, b2_ref, out_ref):
    h = pl.program_id(1)

    @pl.when(h == 0)
    def _():
        out_ref[...] = jnp.broadcast_to(b2_ref[0], out_ref.shape)

    hpart = jnp.maximum(
        jnp.dot(xs_ref[...], w1_ref[0], preferred_element_type=jnp.float32)
        + b1_ref[0], 0.0)
    out_ref[...] += jnp.dot(hpart, w2_ref[0],
                            preferred_element_type=jnp.float32)


def _e_of(b, s_ref):
    val = b * R
    e = jnp.int32(0)
    for ee in range(1, E):
        e += jnp.where(val >= s_ref[ee], 1, 0).astype(jnp.int32)
    return e


def _gmm(gs, Xs, W1, b1, W2, b2):
    grid = (NBLK, H // HT)
    grid_spec = pltpu.PrefetchScalarGridSpec(
        num_scalar_prefetch=1,
        grid=grid,
        in_specs=[
            pl.BlockSpec((R, D), lambda b, h, s: (b, 0)),
            pl.BlockSpec((1, D, HT), lambda b, h, s: (_e_of(b, s), 0, h)),
            pl.BlockSpec((1, 1, HT), lambda b, h, s: (_e_of(b, s), 0, h)),
            pl.BlockSpec((1, HT, D), lambda b, h, s: (_e_of(b, s), h, 0)),
            pl.BlockSpec((1, 1, D), lambda b, h, s: (_e_of(b, s), 0, 0)),
        ],
        out_specs=pl.BlockSpec((R, D), lambda b, h, s: (b, 0)),
    )
    return pl.pallas_call(
        _gmm_kernel,
        grid_spec=grid_spec,
        out_shape=jax.ShapeDtypeStruct((PADN, D), jnp.float32),
    )(gs, Xs, W1, b1.reshape(E, 1, H), W2, b2.reshape(E, 1, D))


def _combine_body(ys_hbm, dest_hbm, w_hbm, out_hbm,
                  dest_v, w_v, ybuf_v, obuf_v, sem0, sem1):
    wid = lax.axis_index("s") * NC + lax.axis_index("c")
    base = wid * CHUNK
    tok0 = wid * (N // NW)
    sems = (sem0, sem1)
    nchunk = CHUNK // 16

    pltpu.sync_copy(dest_hbm.at[pl.ds(base, CHUNK)], dest_v)
    pltpu.sync_copy(w_hbm.at[pl.ds(base, CHUNK)], w_v)

    def gth(cc, b):
        return pltpu.make_async_copy(
            ys_hbm.at[dest_v.at[pl.ds(cc * 16, 16)]], ybuf_v.at[b], sems[b])

    gth(0, 0).start()
    gth(1, 1).start()

    def chunk_step(g, _):
        for b in range(2):
            cc = g * 2 + b
            gth(cc, b).wait()
            @pl.when(cc + 2 < nchunk)
            def _():
                gth(cc + 2, b).start()
            for t in range(8):
                w0 = plsc.load_gather(w_v, [jnp.zeros((L,), jnp.int32)
                                            + (cc * 16 + 2 * t)])
                w1 = plsc.load_gather(w_v, [jnp.zeros((L,), jnp.int32)
                                            + (cc * 16 + 2 * t + 1)])
                for q in range(D // L):
                    y0 = ybuf_v[b, 2 * t, pl.ds(q * L, L)]
                    y1 = ybuf_v[b, 2 * t + 1, pl.ds(q * L, L)]
                    obuf_v[t, pl.ds(q * L, L)] = w0 * y0 + w1 * y1
            pltpu.sync_copy(obuf_v, out_hbm.at[pl.ds(tok0 + cc * 8, 8)])
        return 0

    lax.fori_loop(0, nchunk // 2, chunk_step, 0)


def _combine(Ys, dest, wflat):
    mesh = plsc.VectorSubcoreMesh(core_axis_name="c", subcore_axis_name="s")
    f = pl.kernel(
        _combine_body,
        out_type=jax.ShapeDtypeStruct((N, D), jnp.float32),
        mesh=mesh,
        compiler_params=pltpu.CompilerParams(needs_layout_passes=False),
        scratch_types=[
            pltpu.VMEM((CHUNK,), jnp.int32),
            pltpu.VMEM((CHUNK,), jnp.float32),
            pltpu.VMEM((2, 16, D), jnp.float32),
            pltpu.VMEM((8, D), jnp.float32),
            pltpu.SemaphoreType.DMA,
            pltpu.SemaphoreType.DMA,
        ],
    )
    return f(Ys, dest, wflat)


def kernel(x, Wg, bg, W1, b1, W2, b2):
    B, S, _ = x.shape
    x2d = x.reshape(N, D)
    topw, topi = _gate(x2d, Wg, bg)
    Xs, dest, gs = _dispatch(topi.reshape(A), x2d)
    Ys = _gmm(gs, Xs, W1, b1, W2, b2)
    out = _combine(Ys, dest, topw.reshape(A))
    return out.reshape(B, S, D)


# speedup vs baseline: 1.1085x; 1.0098x over previous
import functools

import jax
import jax.numpy as jnp
from jax import lax
from jax.experimental import pallas as pl
from jax.experimental.pallas import tpu as pltpu
from jax.experimental.pallas import tpu_sc as plsc

N = 4096
D = 1024
H = 4096
E = 8
TOPK = 2
A = N * TOPK
R = 512
PADN = A + E * R
NBLK = PADN // R
HT = 2048

NC, NS, L = 2, 16, 16
NW = NC * NS
CHUNK = A // NW


def _gate_kernel(x_ref, wg_ref, bg_ref, w_ref, i_ref):
    scores = jnp.dot(x_ref[...], wg_ref[...],
                     preferred_element_type=jnp.float32) + bg_ref[...]
    lane = jax.lax.broadcasted_iota(jnp.int32, scores.shape, 1)
    m1 = jnp.max(scores, axis=1, keepdims=True)
    a1 = jnp.argmax(scores, axis=1).reshape(-1, 1)
    masked = jnp.where(lane == a1, -jnp.inf, scores)
    m2 = jnp.max(masked, axis=1, keepdims=True)
    a2 = jnp.argmax(masked, axis=1).reshape(-1, 1)
    z = jnp.exp(m2 - m1)
    w_ref[...] = jnp.concatenate([1.0 / (1.0 + z), z / (1.0 + z)], axis=1)
    i_ref[...] = jnp.concatenate([a1, a2], axis=1)


def _gate(x2d, Wg, bg):
    TB = 1024
    return pl.pallas_call(
        _gate_kernel,
        grid=(N // TB,),
        in_specs=[
            pl.BlockSpec((TB, D), lambda t: (t, 0)),
            pl.BlockSpec((D, E), lambda t: (0, 0)),
            pl.BlockSpec((E,), lambda t: (0,)),
        ],
        out_specs=[
            pl.BlockSpec((TB, TOPK), lambda t: (t, 0)),
            pl.BlockSpec((TB, TOPK), lambda t: (t, 0)),
        ],
        out_shape=[
            jax.ShapeDtypeStruct((N, TOPK), jnp.float32),
            jax.ShapeDtypeStruct((N, TOPK), jnp.int32),
        ],
    )(x2d, Wg, bg)


def _dispatch_body(idx_hbm, x_hbm, xs_hbm, dest_hbm, gs_hbm,
                   idx_v, cnt_v, gs_v, dchunk_v, tok_v, rows_v, sem):
    wid = lax.axis_index("s") * NC + lax.axis_index("c")
    base = wid * CHUNK
    lanes = lax.iota(jnp.int32, L)
    onesf = jnp.ones((L,), jnp.int32)

    pltpu.sync_copy(idx_hbm, idx_v)

    def hist_step(j, carry):
        tot, pre = carry
        v = idx_v[pl.ds(j * L, L)]
        h = jnp.zeros((L,), jnp.int32)
        for e in range(E):
            c = jnp.sum(jnp.where(v == e, 1, 0).astype(jnp.int32))
            h = jnp.where(lanes == e, c, h)
        inpre = jnp.where(j < wid * (CHUNK // L), 1, 0)
        return tot + h, pre + h * inpre

    tot, pre = lax.fori_loop(
        0, A // L, hist_step,
        (jnp.zeros((L,), jnp.int32), jnp.zeros((L,), jnp.int32)))

    padded = (tot + (R - 1)) & ~(R - 1)
    gs = plsc.cumsum(padded) - padded
    gs_v[...] = gs
    cnt_v[...] = gs + pre

    @pl.when(wid == 0)
    def _():
        pltpu.sync_copy(gs_v, gs_hbm)

    for j in range(CHUNK // L):
        v = idx_v[pl.ds(base + j * L, L)]
        rank = jnp.zeros((L,), jnp.int32)
        add = jnp.zeros((L,), jnp.int32)
        for e in range(E):
            m = v == e
            mi = jnp.where(m, 1, 0).astype(jnp.int32)
            s = plsc.cumsum(mi)
            rank = jnp.where(m, s - 1, rank)
            add = jnp.where(lanes == e, jnp.sum(mi), add)
        dest = plsc.load_gather(cnt_v, [v]) + rank
        cnt_v[...] = cnt_v[...] + add
        dchunk_v[j // 4, pl.ds((j % 4) * L, L)] = dest

    for c in range(4):
        pltpu.sync_copy(dchunk_v.at[c], dest_hbm.at[pl.ds(base + c * 64, 64)])
        for j in range(4):
            a0 = base + c * 64 + j * L
            tok_v[pl.ds(j * L, L)] = (a0 + lanes) // TOPK
        pltpu.async_copy(x_hbm.at[tok_v], rows_v, sem).wait()
        pltpu.async_copy(rows_v, xs_hbm.at[dchunk_v.at[c]], sem).wait()


def _dispatch(idxflat, x2d):
    mesh = plsc.VectorSubcoreMesh(core_axis_name="c", subcore_axis_name="s")
    f = pl.kernel(
        _dispatch_body,
        out_type=(
            jax.ShapeDtypeStruct((PADN, D), jnp.float32),
            jax.ShapeDtypeStruct((A,), jnp.int32),
            jax.ShapeDtypeStruct((L,), jnp.int32),
        ),
        mesh=mesh,
        compiler_params=pltpu.CompilerParams(needs_layout_passes=False),
        scratch_types=[
            pltpu.VMEM((A,), jnp.int32),
            pltpu.VMEM((L,), jnp.int32),
            pltpu.VMEM((L,), jnp.int32),
            pltpu.VMEM((4, 64), jnp.int32),
            pltpu.VMEM((64,), jnp.int32),
            pltpu.VMEM((64, D), jnp.float32),
            pltpu.SemaphoreType.DMA,
        ],
    )
    return f(idxflat, x2d)


def _gmm_kernel(s_ref, xs_ref, w1_ref, b1_ref, w2_ref, b2_ref, out_ref):
    h = pl.program_id(1)

    @pl.when(h == 0)
    def _():
        out_ref[...] = jnp.broadcast_to(b2_ref[0], out_ref.shape)

    hpart = jnp.maximum(
        jnp.dot(xs_ref[...], w1_ref[0], preferred_element_type=jnp.float32)
        + b1_ref[0], 0.0)
    out_ref[...] += jnp.dot(hpart, w2_ref[0],
                            preferred_element_type=jnp.float32)


def _e_of(b, s_ref):
    val = b * R
    e = jnp.int32(0)
    for ee in range(1, E):
        e += jnp.where(val >= s_ref[ee], 1, 0).astype(jnp.int32)
    return e


def _gmm(gs, Xs, W1, b1, W2, b2):
    grid = (NBLK, H // HT)
    grid_spec = pltpu.PrefetchScalarGridSpec(
        num_scalar_prefetch=1,
        grid=grid,
        in_specs=[
            pl.BlockSpec((R, D), lambda b, h, s: (b, 0)),
            pl.BlockSpec((1, D, HT), lambda b, h, s: (_e_of(b, s), 0, h)),
            pl.BlockSpec((1, 1, HT), lambda b, h, s: (_e_of(b, s), 0, h)),
            pl.BlockSpec((1, HT, D), lambda b, h, s: (_e_of(b, s), h, 0)),
            pl.BlockSpec((1, 1, D), lambda b, h, s: (_e_of(b, s), 0, 0)),
        ],
        out_specs=pl.BlockSpec((R, D), lambda b, h, s: (b, 0)),
    )
    return pl.pallas_call(
        _gmm_kernel,
        grid_spec=grid_spec,
        out_shape=jax.ShapeDtypeStruct((PADN, D), jnp.float32),
    )(gs, Xs, W1, b1.reshape(E, 1, H), W2, b2.reshape(E, 1, D))


def _combine_body(ys_hbm, dest_hbm, w_hbm, out_hbm,
                  dest_v, w_v, ybuf_v, obuf_v, sem0, sem1):
    wid = lax.axis_index("s") * NC + lax.axis_index("c")
    base = wid * CHUNK
    tok0 = wid * (N // NW)
    sems = (sem0, sem1)
    nchunk = CHUNK // 16

    pltpu.sync_copy(dest_hbm.at[pl.ds(base, CHUNK)], dest_v)
    pltpu.sync_copy(w_hbm.at[pl.ds(base, CHUNK)], w_v)

    def gth(cc, b):
        return pltpu.make_async_copy(
            ys_hbm.at[dest_v.at[pl.ds(cc * 16, 16)]], ybuf_v.at[b], sems[b])

    gth(0, 0).start()
    gth(1, 1).start()

    def chunk_step(g, _):
        for b in range(2):
            cc = g * 2 + b
            gth(cc, b).wait()
            for t in range(8):
                w0 = plsc.load_gather(w_v, [jnp.zeros((L,), jnp.int32)
                                            + (cc * 16 + 2 * t)])
                w1 = plsc.load_gather(w_v, [jnp.zeros((L,), jnp.int32)
                                            + (cc * 16 + 2 * t + 1)])
                for q in range(D // L):
                    y0 = ybuf_v[b, 2 * t, pl.ds(q * L, L)]
                    y1 = ybuf_v[b, 2 * t + 1, pl.ds(q * L, L)]
                    obuf_v[t, pl.ds(q * L, L)] = w0 * y0 + w1 * y1
            pltpu.sync_copy(obuf_v, out_hbm.at[pl.ds(tok0 + cc * 8, 8)])

            @pl.when(cc + 2 < nchunk)
            def _():
                gth(cc + 2, b).start()
        return 0

    lax.fori_loop(0, nchunk // 2, chunk_step, 0)


def _combine(Ys, dest, wflat):
    mesh = plsc.VectorSubcoreMesh(core_axis_name="c", subcore_axis_name="s")
    f = pl.kernel(
        _combine_body,
        out_type=jax.ShapeDtypeStruct((N, D), jnp.float32),
        mesh=mesh,
        compiler_params=pltpu.CompilerParams(needs_layout_passes=False),
        scratch_types=[
            pltpu.VMEM((CHUNK,), jnp.int32),
            pltpu.VMEM((CHUNK,), jnp.float32),
            pltpu.VMEM((2, 16, D), jnp.float32),
            pltpu.VMEM((8, D), jnp.float32),
            pltpu.SemaphoreType.DMA,
            pltpu.SemaphoreType.DMA,
        ],
    )
    return f(Ys, dest, wflat)


def kernel(x, Wg, bg, W1, b1, W2, b2):
    B, S, _ = x.shape
    x2d = x.reshape(N, D)
    topw, topi = _gate(x2d, Wg, bg)
    Xs, dest, gs = _dispatch(topi.reshape(A), x2d)
    Ys = _gmm(gs, Xs, W1, b1, W2, b2)
    out = _combine(Ys, dest, topw.reshape(A))
    return out.reshape(B, S, D)


# speedup vs baseline: 1.1194x; 1.0098x over previous
import functools

import jax
import jax.numpy as jnp
from jax import lax
from jax.experimental import pallas as pl
from jax.experimental.pallas import tpu as pltpu
from jax.experimental.pallas import tpu_sc as plsc

N = 4096
D = 1024
H = 4096
E = 8
TOPK = 2
A = N * TOPK
R = 512
PADN = A + E * R
NBLK = PADN // R
HT = 2048

NC, NS, L = 2, 16, 16
NW = NC * NS
CHUNK = A // NW


def _gate_kernel(x_ref, wg_ref, bg_ref, w_ref, i_ref):
    scores = jnp.dot(x_ref[...], wg_ref[...],
                     preferred_element_type=jnp.float32) + bg_ref[...]
    lane = jax.lax.broadcasted_iota(jnp.int32, scores.shape, 1)
    m1 = jnp.max(scores, axis=1, keepdims=True)
    a1 = jnp.argmax(scores, axis=1).reshape(-1, 1)
    masked = jnp.where(lane == a1, -jnp.inf, scores)
    m2 = jnp.max(masked, axis=1, keepdims=True)
    a2 = jnp.argmax(masked, axis=1).reshape(-1, 1)
    z = jnp.exp(m2 - m1)
    w_ref[...] = jnp.concatenate([1.0 / (1.0 + z), z / (1.0 + z)], axis=1)
    i_ref[...] = jnp.concatenate([a1, a2], axis=1)


def _gate(x2d, Wg, bg):
    TB = 1024
    return pl.pallas_call(
        _gate_kernel,
        grid=(N // TB,),
        in_specs=[
            pl.BlockSpec((TB, D), lambda t: (t, 0)),
            pl.BlockSpec((D, E), lambda t: (0, 0)),
            pl.BlockSpec((E,), lambda t: (0,)),
        ],
        out_specs=[
            pl.BlockSpec((TB, TOPK), lambda t: (t, 0)),
            pl.BlockSpec((TB, TOPK), lambda t: (t, 0)),
        ],
        out_shape=[
            jax.ShapeDtypeStruct((N, TOPK), jnp.float32),
            jax.ShapeDtypeStruct((N, TOPK), jnp.int32),
        ],
    )(x2d, Wg, bg)


def _dispatch_body(idx_hbm, x_hbm, xs_hbm, dest_hbm, gs_hbm,
                   idx_v, cnt_v, gs_v, dchunk_v, dflat_v, tok_v, bufs_v,
                   sg0, sg1, sg2, ss0, ss1, ss2):
    wid = lax.axis_index("s") * NC + lax.axis_index("c")
    base = wid * CHUNK
    lanes = lax.iota(jnp.int32, L)
    sg = (sg0, sg1, sg2)
    ss = (ss0, ss1, ss2)
    NCH = 8

    pltpu.sync_copy(idx_hbm, idx_v)

    for c in range(NCH):
        for j in range(2):
            a0 = base + c * 32 + j * L
            tok_v[c, pl.ds(j * L, L)] = (a0 + lanes) // TOPK

    def gth(c):
        return pltpu.make_async_copy(
            x_hbm.at[tok_v.at[c]], bufs_v.at[c % 3], sg[c % 3])

    def sct(c):
        return pltpu.make_async_copy(
            bufs_v.at[c % 3], xs_hbm.at[dchunk_v.at[c]], ss[c % 3])

    gth(0).start()
    gth(1).start()
    gth(2).start()

    def hist_step(j, carry):
        tot, pre = carry
        v = idx_v[pl.ds(j * L, L)]
        h = jnp.zeros((L,), jnp.int32)
        for e in range(E):
            c = jnp.sum(jnp.where(v == e, 1, 0).astype(jnp.int32))
            h = jnp.where(lanes == e, c, h)
        inpre = jnp.where(j < wid * (CHUNK // L), 1, 0)
        return tot + h, pre + h * inpre

    tot, pre = lax.fori_loop(
        0, A // L, hist_step,
        (jnp.zeros((L,), jnp.int32), jnp.zeros((L,), jnp.int32)))

    padded = (tot + (R - 1)) & ~(R - 1)
    gs = plsc.cumsum(padded) - padded
    gs_v[...] = gs
    cnt_v[...] = gs + pre

    @pl.when(wid == 0)
    def _():
        pltpu.sync_copy(gs_v, gs_hbm)

    for j in range(CHUNK // L):
        v = idx_v[pl.ds(base + j * L, L)]
        rank = jnp.zeros((L,), jnp.int32)
        add = jnp.zeros((L,), jnp.int32)
        for e in range(E):
            m = v == e
            mi = jnp.where(m, 1, 0).astype(jnp.int32)
            s = plsc.cumsum(mi)
            rank = jnp.where(m, s - 1, rank)
            add = jnp.where(lanes == e, jnp.sum(mi), add)
        dest = plsc.load_gather(cnt_v, [v]) + rank
        cnt_v[...] = cnt_v[...] + add
        dchunk_v[j // 2, pl.ds((j % 2) * L, L)] = dest
        dflat_v[pl.ds(j * L, L)] = dest

    pltpu.sync_copy(dflat_v, dest_hbm.at[pl.ds(base, CHUNK)])

    for c in range(NCH):
        if c >= 3:
            sct(c - 3).wait()
            gth(c).start()
        gth(c).wait()
        sct(c).start()
    for c in range(NCH - 3, NCH):
        sct(c).wait()


def _dispatch(idxflat, x2d):
    mesh = plsc.VectorSubcoreMesh(core_axis_name="c", subcore_axis_name="s")
    f = pl.kernel(
        _dispatch_body,
        out_type=(
            jax.ShapeDtypeStruct((PADN, D), jnp.float32),
            jax.ShapeDtypeStruct((A,), jnp.int32),
            jax.ShapeDtypeStruct((L,), jnp.int32),
        ),
        mesh=mesh,
        compiler_params=pltpu.CompilerParams(needs_layout_passes=False),
        scratch_types=[
            pltpu.VMEM((A,), jnp.int32),
            pltpu.VMEM((L,), jnp.int32),
            pltpu.VMEM((L,), jnp.int32),
            pltpu.VMEM((8, 32), jnp.int32),
            pltpu.VMEM((CHUNK,), jnp.int32),
            pltpu.VMEM((8, 32), jnp.int32),
            pltpu.VMEM((3, 32, D), jnp.float32),
            pltpu.SemaphoreType.DMA,
            pltpu.SemaphoreType.DMA,
            pltpu.SemaphoreType.DMA,
            pltpu.SemaphoreType.DMA,
            pltpu.SemaphoreType.DMA,
            pltpu.SemaphoreType.DMA,
        ],
    )
    return f(idxflat, x2d)


def _gmm_kernel(s_ref, xs_ref, w1_ref, b1_ref, w2_ref, b2_ref, out_ref):
    h = pl.program_id(1)

    @pl.when(h == 0)
    def _():
        out_ref[...] = jnp.broadcast_to(b2_ref[0], out_ref.shape)

    hpart = jnp.maximum(
        jnp.dot(xs_ref[...], w1_ref[0], preferred_element_type=jnp.float32)
        + b1_ref[0], 0.0)
    out_ref[...] += jnp.dot(hpart, w2_ref[0],
                            preferred_element_type=jnp.float32)


def _e_of(b, s_ref):
    val = b * R
    e = jnp.int32(0)
    for ee in range(1, E):
        e += jnp.where(val >= s_ref[ee], 1, 0).astype(jnp.int32)
    return e


def _gmm(gs, Xs, W1, b1, W2, b2):
    grid = (NBLK, H // HT)
    grid_spec = pltpu.PrefetchScalarGridSpec(
        num_scalar_prefetch=1,
        grid=grid,
        in_specs=[
            pl.BlockSpec((R, D), lambda b, h, s: (b, 0)),
            pl.BlockSpec((1, D, HT), lambda b, h, s: (_e_of(b, s), 0, h)),
            pl.BlockSpec((1, 1, HT), lambda b, h, s: (_e_of(b, s), 0, h)),
            pl.BlockSpec((1, HT, D), lambda b, h, s: (_e_of(b, s), h, 0)),
            pl.BlockSpec((1, 1, D), lambda b, h, s: (_e_of(b, s), 0, 0)),
        ],
        out_specs=pl.BlockSpec((R, D), lambda b, h, s: (b, 0)),
    )
    return pl.pallas_call(
        _gmm_kernel,
        grid_spec=grid_spec,
        out_shape=jax.ShapeDtypeStruct((PADN, D), jnp.float32),
    )(gs, Xs, W1, b1.reshape(E, 1, H), W2, b2.reshape(E, 1, D))


def _combine_body(ys_hbm, dest_hbm, w_hbm, out_hbm,
                  dest_v, w_v, ybuf_v, obuf_v, sem0, sem1):
    wid = lax.axis_index("s") * NC + lax.axis_index("c")
    base = wid * CHUNK
    tok0 = wid * (N // NW)
    sems = (sem0, sem1)
    nchunk = CHUNK // 16

    pltpu.sync_copy(dest_hbm.at[pl.ds(base, CHUNK)], dest_v)
    pltpu.sync_copy(w_hbm.at[pl.ds(base, CHUNK)], w_v)

    def gth(cc, b):
        return pltpu.make_async_copy(
            ys_hbm.at[dest_v.at[pl.ds(cc * 16, 16)]], ybuf_v.at[b], sems[b])

    gth(0, 0).start()
    gth(1, 1).start()

    def chunk_step(g, _):
        for b in range(2):
            cc = g * 2 + b
            gth(cc, b).wait()
            for t in range(8):
                w0 = plsc.load_gather(w_v, [jnp.zeros((L,), jnp.int32)
                                            + (cc * 16 + 2 * t)])
                w1 = plsc.load_gather(w_v, [jnp.zeros((L,), jnp.int32)
                                            + (cc * 16 + 2 * t + 1)])
                for q in range(D // L):
                    y0 = ybuf_v[b, 2 * t, pl.ds(q * L, L)]
                    y1 = ybuf_v[b, 2 * t + 1, pl.ds(q * L, L)]
                    obuf_v[t, pl.ds(q * L, L)] = w0 * y0 + w1 * y1
            pltpu.sync_copy(obuf_v, out_hbm.at[pl.ds(tok0 + cc * 8, 8)])

            @pl.when(cc + 2 < nchunk)
            def _():
                gth(cc + 2, b).start()
        return 0

    lax.fori_loop(0, nchunk // 2, chunk_step, 0)


def _combine(Ys, dest, wflat):
    mesh = plsc.VectorSubcoreMesh(core_axis_name="c", subcore_axis_name="s")
    f = pl.kernel(
        _combine_body,
        out_type=jax.ShapeDtypeStruct((N, D), jnp.float32),
        mesh=mesh,
        compiler_params=pltpu.CompilerParams(needs_layout_passes=False),
        scratch_types=[
            pltpu.VMEM((CHUNK,), jnp.int32),
            pltpu.VMEM((CHUNK,), jnp.float32),
            pltpu.VMEM((2, 16, D), jnp.float32),
            pltpu.VMEM((8, D), jnp.float32),
            pltpu.SemaphoreType.DMA,
            pltpu.SemaphoreType.DMA,
        ],
    )
    return f(Ys, dest, wflat)


def kernel(x, Wg, bg, W1, b1, W2, b2):
    B, S, _ = x.shape
    x2d = x.reshape(N, D)
    topw, topi = _gate(x2d, Wg, bg)
    Xs, dest, gs = _dispatch(topi.reshape(A), x2d)
    Ys = _gmm(gs, Xs, W1, b1, W2, b2)
    out = _combine(Ys, dest, topw.reshape(A))
    return out.reshape(B, S, D)
